# baseline (speedup 1.0000x reference)
import jax
import jax.numpy as jnp
from jax.experimental import pallas as pl
from jax.experimental.pallas import tpu as pltpu

_NUM_MODES = 65536
_DIM = 128
_BATCH = 1024
_BK = 4096
_NKB = _NUM_MODES // _BK


_LOG2E = 1.4426950408889634


def _body(q_ref, k_ref, v_ref, attn_ref, out_ref, acc_ref, l_ref, qn_ref):
    p = pl.program_id(1)
    kb = pl.program_id(2)

    @pl.when((p == 0) & (kb == 0))
    def _init():
        q = q_ref[...]
        qn_ref[...] = (
            q * (_LOG2E * jax.lax.rsqrt(jnp.maximum(jnp.sum(q * q, axis=1, keepdims=True), 1e-24)))
        ).astype(jnp.bfloat16)
        acc_ref[...] = jnp.zeros_like(acc_ref)
        l_ref[...] = jnp.zeros_like(l_ref)

    k = k_ref[...]
    kn = (k * jax.lax.rsqrt(jnp.maximum(jnp.sum(k * k, axis=1, keepdims=True), 1e-24))).astype(jnp.bfloat16)

    @pl.when(p == 0)
    def _pass0():
        sims2 = jax.lax.dot_general(
            qn_ref[...], kn, (((1,), (1,)), ((), ())),
            preferred_element_type=jnp.float32,
        )
        e = jnp.exp2(sims2 - _LOG2E)
        l_ref[...] = l_ref[...] + jnp.sum(e, axis=1, keepdims=True)

        @pl.when(kb == _NKB - 1)
        def _fold():
            l_ref[...] = jnp.log2(l_ref[...]) + _LOG2E

    @pl.when(p == 1)
    def _pass1():
        sims2 = jax.lax.dot_general(
            qn_ref[...], kn, (((1,), (1,)), ((), ())),
            preferred_element_type=jnp.float32,
        )
        en = jnp.exp2(sims2 - l_ref[:, 0:1])
        attn_ref[...] = en
        acc_ref[...] = acc_ref[...] + jnp.dot(
            en.astype(jnp.bfloat16), v_ref[...].astype(jnp.bfloat16),
            preferred_element_type=jnp.float32,
        )

        @pl.when(kb == _NKB - 1)
        def _final():
            out_ref[...] = acc_ref[...]


@jax.jit
def kernel(query, keys, values):
    attn, readout = pl.pallas_call(
        _body,
        grid=(2, 2, _NKB),
        in_specs=[
            pl.BlockSpec((_BATCH // 2, _DIM), lambda h, p, k: (h, 0)),
            pl.BlockSpec((_BK, _DIM), lambda h, p, k: (k, 0)),
            pl.BlockSpec((_BK, _DIM), lambda h, p, k: (jax.lax.select(p == 1, k, 0), 0)),
        ],
        out_specs=[
            pl.BlockSpec((_BATCH // 2, _BK), lambda h, p, k: (h, jax.lax.select(p == 0, 0, k))),
            pl.BlockSpec((_BATCH // 2, _DIM), lambda h, p, k: (h, 0)),
        ],
        out_shape=[
            jax.ShapeDtypeStruct((_BATCH, _NUM_MODES), jnp.float32),
            jax.ShapeDtypeStruct((_BATCH, _DIM), jnp.float32),
        ],
        scratch_shapes=[
            pltpu.VMEM((_BATCH // 2, _DIM), jnp.float32),
            pltpu.VMEM((_BATCH // 2, 128), jnp.float32),
            pltpu.VMEM((_BATCH // 2, _DIM), jnp.bfloat16),
        ],
        compiler_params=pltpu.CompilerParams(
            dimension_semantics=("parallel", "arbitrary", "arbitrary"),
            vmem_limit_bytes=63 * 1024 * 1024,
        ),
    )(query, keys, values)
    return (readout, attn)


# speedup vs baseline: 1.2724x; 1.2724x over previous
import jax
import jax.numpy as jnp
from jax.experimental import pallas as pl
from jax.experimental.pallas import tpu as pltpu

_NUM_MODES = 65536
_DIM = 128
_BATCH = 1024
_BK = 4096
_NKB = _NUM_MODES // _BK


_LOG2E = 1.4426950408889634


def _body(q_ref, k_ref, v_ref, attn_ref, out_ref, acc_ref, l_ref, qn_ref):
    p = pl.program_id(0)
    kb = pl.program_id(1)

    @pl.when((p == 0) & (kb == 0))
    def _init():
        q = q_ref[...]
        qn_ref[...] = (
            q * (_LOG2E * jax.lax.rsqrt(jnp.maximum(jnp.sum(q * q, axis=1, keepdims=True), 1e-24)))
        ).astype(jnp.bfloat16)
        acc_ref[...] = jnp.zeros_like(acc_ref)
        l_ref[...] = jnp.zeros_like(l_ref)

    k = k_ref[...]
    kn = (k * jax.lax.rsqrt(jnp.maximum(jnp.sum(k * k, axis=1, keepdims=True), 1e-24))).astype(jnp.bfloat16)

    @pl.when(p == 0)
    def _pass0():
        sims2 = jax.lax.dot_general(
            qn_ref[...], kn, (((1,), (1,)), ((), ())),
            preferred_element_type=jnp.float32,
        )
        e = jnp.exp2(sims2 - _LOG2E)
        l_ref[...] = l_ref[...] + jnp.sum(e, axis=1, keepdims=True)

        @pl.when(kb == _NKB - 1)
        def _fold():
            l_ref[...] = jnp.log2(l_ref[...]) + _LOG2E

    @pl.when(p == 1)
    def _pass1():
        sims2 = jax.lax.dot_general(
            qn_ref[...], kn, (((1,), (1,)), ((), ())),
            preferred_element_type=jnp.float32,
        )
        en = jnp.exp2(sims2 - l_ref[:, 0:1])
        attn_ref[...] = en
        acc_ref[...] = acc_ref[...] + jnp.dot(
            en.astype(jnp.bfloat16), v_ref[...].astype(jnp.bfloat16),
            preferred_element_type=jnp.float32,
        )

        @pl.when(kb == _NKB - 1)
        def _final():
            out_ref[...] = acc_ref[...]


@jax.jit
def kernel(query, keys, values):
    attn, readout = pl.pallas_call(
        _body,
        grid=(2, _NKB),
        in_specs=[
            pl.BlockSpec((_BATCH, _DIM), lambda p, k: (0, 0)),
            pl.BlockSpec((_BK, _DIM), lambda p, k: (k, 0)),
            pl.BlockSpec((_BK, _DIM), lambda p, k: (jax.lax.select(p == 1, k, 0), 0)),
        ],
        out_specs=[
            pl.BlockSpec((_BATCH, _BK), lambda p, k: (0, jax.lax.select(p == 0, 0, k))),
            pl.BlockSpec((_BATCH, _DIM), lambda p, k: (0, 0)),
        ],
        out_shape=[
            jax.ShapeDtypeStruct((_BATCH, _NUM_MODES), jnp.float32),
            jax.ShapeDtypeStruct((_BATCH, _DIM), jnp.float32),
        ],
        scratch_shapes=[
            pltpu.VMEM((_BATCH, _DIM), jnp.float32),
            pltpu.VMEM((_BATCH, 128), jnp.float32),
            pltpu.VMEM((_BATCH, _DIM), jnp.bfloat16),
        ],
        compiler_params=pltpu.CompilerParams(
            dimension_semantics=("arbitrary", "arbitrary"),
            vmem_limit_bytes=63 * 1024 * 1024,
        ),
    )(query, keys, values)
    return (readout, attn)
